# Initial kernel scaffold; baseline (speedup 1.0000x reference)
import functools

import jax
import jax.numpy as jnp
from jax import lax
from jax.experimental import pallas as pl
from jax.experimental.pallas import tpu as pltpu
from jax.experimental.pallas import tpu_sc as plsc

NC = 2
NS = 16
NW = NC * NS
L = 16
K = 128


def _make_deg_kernel(n, ch):
    mesh = plsc.VectorSubcoreMesh(core_axis_name="c", subcore_axis_name="s")

    @functools.partial(
        pl.kernel,
        mesh=mesh,
        out_type=jax.ShapeDtypeStruct((NW, n), jnp.float32),
        scratch_types=[
            pltpu.VMEM((ch, K), jnp.int32),
            pltpu.VMEM((ch, K), jnp.float32),
            pltpu.VMEM((n,), jnp.float32),
        ],
    )
    def deg_k(col_hbm, ew_hbm, degp_hbm, col_l, ew_l, deg_l):
        cid = lax.axis_index("c")
        sid = lax.axis_index("s")
        wid = cid * NS + sid
        z = jnp.zeros((L,), jnp.float32)

        @pl.loop(0, n // L, unroll=8)
        def _zero(i):
            deg_l[pl.ds(i * L, L)] = z

        pltpu.sync_copy(col_hbm.at[wid], col_l)
        pltpu.sync_copy(ew_hbm.at[wid], ew_l)

        @pl.loop(0, ch)
        def _edges(g):
            for j in range(K // L):
                c16 = col_l[g, pl.ds(j * L, L)]
                w16 = ew_l[g, pl.ds(j * L, L)]
                plsc.addupdate_scatter(deg_l, [c16], w16)

        pltpu.sync_copy(deg_l, degp_hbm.at[wid])

    return deg_k


def _make_msg_kernel(n, d, ch):
    stripe = n // NS
    n_full = stripe // K
    rem = stripe - n_full * K
    mesh = plsc.VectorSubcoreMesh(core_axis_name="c", subcore_axis_name="s")

    @functools.partial(
        pl.kernel,
        mesh=mesh,
        out_type=jax.ShapeDtypeStruct((NC, n, d), jnp.float32),
        scratch_types=[
            pltpu.VMEM((ch, K), jnp.int32),
            pltpu.VMEM((ch, K), jnp.int32),
            pltpu.VMEM((ch, K), jnp.float32),
            pltpu.VMEM((K, d), jnp.float32),
            pltpu.VMEM((K, d), jnp.float32),
            pltpu.VMEM_SHARED((n, d), jnp.float32),
            pltpu.SemaphoreType.DMA,
            pltpu.SemaphoreType.DMA,
        ],
    )
    def msg_k(row_hbm, col_hbm, ew_hbm, g_hbm, p_hbm,
              row_l, col_l, ew_l, b0, b1, acc, s0, s1):
        cid = lax.axis_index("c")
        sid = lax.axis_index("s")
        wid = cid * NS + sid
        z = jnp.zeros((L,), jnp.float32)

        @pl.loop(0, K, unroll=4)
        def _zero(e):
            for j in range(d // L):
                b0[e, pl.ds(j * L, L)] = z

        base = sid * stripe
        for t in range(n_full):
            pltpu.sync_copy(b0, acc.at[pl.ds(base + t * K, K)])
        if rem:
            pltpu.sync_copy(b0.at[pl.ds(0, rem)],
                            acc.at[pl.ds(base + n_full * K, rem)])
        plsc.subcore_barrier()

        pltpu.sync_copy(row_hbm.at[wid], row_l)
        pltpu.sync_copy(col_hbm.at[wid], col_l)
        pltpu.sync_copy(ew_hbm.at[wid], ew_l)

        def scale(buf, g):
            @pl.loop(0, K, unroll=8)
            def _rows(e):
                cval = ew_l[g, e]
                for j in range(d // L):
                    buf[e, pl.ds(j * L, L)] = buf[e, pl.ds(j * L, L)] * cval

        pltpu.async_copy(g_hbm.at[row_l.at[0]], b0, s0)

        @pl.loop(0, ch, step=2)
        def _pair(g):
            pltpu.async_copy(g_hbm.at[row_l.at[g + 1]], b1, s1)
            pltpu.make_async_copy(g_hbm.at[row_l.at[g]], b0, s0).wait()
            scale(b0, g)
            pltpu.sync_copy(b0, acc.at[col_l.at[g]], add=True)

            @pl.when(g + 2 < ch)
            def _():
                pltpu.async_copy(g_hbm.at[row_l.at[g + 2]], b0, s0)

            pltpu.make_async_copy(g_hbm.at[row_l.at[g + 1]], b1, s1).wait()
            scale(b1, g + 1)
            pltpu.sync_copy(b1, acc.at[col_l.at[g + 1]], add=True)

        plsc.subcore_barrier()
        pltpu.sync_copy(acc.at[pl.ds(base, stripe)],
                        p_hbm.at[cid, pl.ds(base, stripe)])

    return msg_k


def _transform_body(degp_ref, x_ref, w_ref, g_ref, dis_ref):
    deg = jnp.sum(degp_ref[...], axis=1, keepdims=True) + 1.0
    dis = lax.rsqrt(deg)
    h = jnp.dot(x_ref[...], w_ref[...], preferred_element_type=jnp.float32)
    g_ref[...] = h * dis
    dis_ref[...] = dis


def _combine_body(p_ref, g_ref, dis_ref, b_ref, o_ref):
    o_ref[...] = dis_ref[...] * (p_ref[0] + p_ref[1] + g_ref[...]) + b_ref[...]


def kernel(x, edge_index, edge_attr, W, b):
    n, d_in = x.shape
    d = W.shape[1]
    e = edge_attr.shape[0]

    epw = NW * K
    ch = -(-e // epw)
    if ch % 2:
        ch += 1
    pad = ch * epw - e
    idt = edge_index.dtype
    row3 = jnp.concatenate(
        [edge_index[0], jnp.zeros((pad,), idt)]).reshape(NW, ch, K)
    col3 = jnp.concatenate(
        [edge_index[1], jnp.zeros((pad,), idt)]).reshape(NW, ch, K)
    ew3 = jnp.concatenate(
        [edge_attr, jnp.zeros((pad,), edge_attr.dtype)]).reshape(NW, ch, K)

    degp = _make_deg_kernel(n, ch)(col3, ew3)
    degp_t = degp.T

    blk = 2000
    g, dis = pl.pallas_call(
        _transform_body,
        grid=(n // blk,),
        in_specs=[
            pl.BlockSpec((blk, NW), lambda i: (i, 0)),
            pl.BlockSpec((blk, d_in), lambda i: (i, 0)),
            pl.BlockSpec((d_in, d), lambda i: (0, 0)),
        ],
        out_specs=[
            pl.BlockSpec((blk, d), lambda i: (i, 0)),
            pl.BlockSpec((blk, 1), lambda i: (i, 0)),
        ],
        out_shape=[
            jax.ShapeDtypeStruct((n, d), jnp.float32),
            jax.ShapeDtypeStruct((n, 1), jnp.float32),
        ],
    )(degp_t, x, W)

    p = _make_msg_kernel(n, d, ch)(row3, col3, ew3, g)

    out = pl.pallas_call(
        _combine_body,
        grid=(n // blk,),
        in_specs=[
            pl.BlockSpec((NC, blk, d), lambda i: (0, i, 0)),
            pl.BlockSpec((blk, d), lambda i: (i, 0)),
            pl.BlockSpec((blk, 1), lambda i: (i, 0)),
            pl.BlockSpec((1, d), lambda i: (0, 0)),
        ],
        out_specs=pl.BlockSpec((blk, d), lambda i: (i, 0)),
        out_shape=jax.ShapeDtypeStruct((n, d), jnp.float32),
    )(p, g, dis, b.reshape(1, d))
    return out


# speedup vs baseline: 15.0335x; 15.0335x over previous
import functools

import jax
import jax.numpy as jnp
from jax import lax
from jax.experimental import pallas as pl
from jax.experimental.pallas import tpu as pltpu
from jax.experimental.pallas import tpu_sc as plsc

NC = 2
NS = 16
NW = NC * NS
L = 16
K = 32


def _stripe_copies(src_get, dst_get, stripe):
    n_full = stripe // K
    rem = stripe - n_full * K
    for t in range(n_full):
        yield src_get(t * K, K), dst_get(t * K, K)
    if rem:
        yield src_get(n_full * K, rem), dst_get(n_full * K, rem)


def _make_deg_kernel(npad, ch_d):
    stripe = npad // NS
    kd = 128
    mesh = plsc.VectorSubcoreMesh(core_axis_name="c", subcore_axis_name="s")

    @functools.partial(
        pl.kernel,
        mesh=mesh,
        out_type=jax.ShapeDtypeStruct((NC, npad, L), jnp.float32),
        compiler_params=pltpu.CompilerParams(use_tc_tiling_on_sc=False),
        scratch_types=[
            pltpu.VMEM((ch_d, kd), jnp.int32),
            pltpu.VMEM((ch_d, kd), jnp.float32),
            pltpu.VMEM((kd, L), jnp.float32),
            pltpu.VMEM_SHARED((npad, L), jnp.float32),
        ],
    )
    def deg_k(col_hbm, ew_hbm, z_hbm, degp_hbm, col_l, ew_l, rb, acc):
        cid = lax.axis_index("c")
        sid = lax.axis_index("s")
        wid = cid * NS + sid
        base = sid * stripe
        pltpu.sync_copy(z_hbm.at[pl.ds(base, stripe)],
                        acc.at[pl.ds(base, stripe)])
        pltpu.sync_copy(col_hbm.at[wid], col_l)
        pltpu.sync_copy(ew_hbm.at[wid], ew_l)
        plsc.subcore_barrier()
        ones = jnp.ones((L,), jnp.float32)

        @pl.loop(0, ch_d)
        def _edges(g):
            for j in range(kd // L):
                ew16 = ew_l[g, pl.ds(j * L, L)]
                for lane in range(L):
                    rb[j * L + lane, pl.ds(0, L)] = ones * ew16[lane]
            pltpu.sync_copy(rb, acc.at[col_l.at[g]], add=True)

        plsc.subcore_barrier()
        pltpu.sync_copy(acc.at[pl.ds(base, stripe)],
                        degp_hbm.at[cid, pl.ds(base, stripe)])

    return deg_k


def _make_msg_kernel(n, npad, d, ch, sh):
    stripe = npad // NS
    msk = (1 << sh) - 1
    mesh = plsc.VectorSubcoreMesh(core_axis_name="c", subcore_axis_name="s")

    @functools.partial(
        pl.kernel,
        mesh=mesh,
        out_type=jax.ShapeDtypeStruct((NC, npad, d), jnp.float32),
        compiler_params=pltpu.CompilerParams(use_tc_tiling_on_sc=False),
        scratch_types=[
            pltpu.VMEM((ch, K), jnp.int32),
            pltpu.VMEM((ch, K), jnp.int32),
            pltpu.VMEM((ch, K), jnp.int32),
            pltpu.VMEM((ch, K), jnp.float32),
            pltpu.VMEM((K, d), jnp.float32),
            pltpu.VMEM((K, d), jnp.float32),
            pltpu.VMEM_SHARED((npad, d), jnp.float32),
            pltpu.SemaphoreType.DMA,
            pltpu.SemaphoreType.DMA,
        ],
    )
    def msg_k(rc_hbm, ew_hbm, g_hbm, p_hbm,
              rc_l, row_l, col_l, ew_l, b0, b1, acc, s0, s1):
        cid = lax.axis_index("c")
        sid = lax.axis_index("s")
        wid = cid * NS + sid
        z = jnp.zeros((L,), jnp.float32)

        @pl.loop(0, K, unroll=4)
        def _zero(e):
            for j in range(d // L):
                b0[e, pl.ds(j * L, L)] = z

        base = sid * stripe
        for off, sz in [(t * K, K) for t in range(stripe // K)] + (
                [(stripe - stripe % K, stripe % K)] if stripe % K else []):
            pltpu.sync_copy(b0.at[pl.ds(0, sz)], acc.at[pl.ds(base + off, sz)])
        plsc.subcore_barrier()

        pltpu.sync_copy(rc_hbm.at[wid], rc_l)
        pltpu.sync_copy(ew_hbm.at[wid], ew_l)

        @pl.loop(0, ch)
        def _unpack(g):
            for j in range(K // L):
                p16 = rc_l[g, pl.ds(j * L, L)]
                row_l[g, pl.ds(j * L, L)] = lax.shift_right_logical(p16, sh)
                col_l[g, pl.ds(j * L, L)] = lax.bitwise_and(p16, msk)

        def scale(buf, g):
            @pl.loop(0, K // L)
            def _grp(i):
                ew16 = ew_l[g, pl.ds(i * L, L)]
                for lane in range(L):
                    cval = ew16[lane]
                    ei = i * L + lane
                    for j in range(d // L):
                        buf[ei, pl.ds(j * L, L)] = (
                            buf[ei, pl.ds(j * L, L)] * cval)

        pltpu.async_copy(g_hbm.at[row_l.at[0]], b0, s0)

        @pl.loop(0, ch, step=2)
        def _pair(g):
            pltpu.async_copy(g_hbm.at[row_l.at[g + 1]], b1, s1)
            pltpu.make_async_copy(g_hbm.at[row_l.at[g]], b0, s0).wait()
            scale(b0, g)
            pltpu.sync_copy(b0, acc.at[col_l.at[g]], add=True)

            @pl.when(g + 2 < ch)
            def _():
                pltpu.async_copy(g_hbm.at[row_l.at[g + 2]], b0, s0)

            pltpu.make_async_copy(g_hbm.at[row_l.at[g + 1]], b1, s1).wait()
            scale(b1, g + 1)
            pltpu.sync_copy(b1, acc.at[col_l.at[g + 1]], add=True)

        plsc.subcore_barrier()
        pltpu.sync_copy(acc.at[pl.ds(base, stripe)],
                        p_hbm.at[cid, pl.ds(base, stripe)])

    return msg_k


def _transform_body(degp_ref, x_ref, w_ref, g_ref, dis_ref):
    deg = jnp.sum(degp_ref[...], axis=1, keepdims=True) + 1.0
    dis = lax.rsqrt(deg)
    h = jnp.dot(x_ref[...], w_ref[...], preferred_element_type=jnp.float32)
    g_ref[...] = h * dis
    dis_ref[...] = dis


def _combine_body(p_ref, g_ref, dis_ref, b_ref, o_ref):
    o_ref[...] = dis_ref[...] * (p_ref[0] + p_ref[1] + g_ref[...]) + b_ref[...]


def kernel(x, edge_index, edge_attr, W, b):
    n, d_in = x.shape
    d = W.shape[1]
    e = edge_attr.shape[0]
    npad = -(-n // (8 * NS)) * (8 * NS)

    epw = NW * K
    ch = -(-e // epw)
    ch += (-ch) % (128 // K)
    if ch % 2:
        ch += 1
    ch_d = ch * K // 128
    pad = ch * epw - e
    idt = edge_index.dtype
    sh = (n - 1).bit_length()
    assert (n << sh) < 2 ** 31
    col3d = jnp.concatenate(
        [edge_index[1], jnp.zeros((pad,), idt)]).reshape(NW, ch_d, 128)
    rc3 = jnp.concatenate(
        [(edge_index[0] << sh) | edge_index[1],
         jnp.zeros((pad,), idt)]).reshape(NW, ch, K)
    ew1 = jnp.concatenate([edge_attr, jnp.zeros((pad,), edge_attr.dtype)])
    ew3 = ew1.reshape(NW, ch, K)
    ew3d = ew1.reshape(NW, ch_d, 128)
    zeros_d = jnp.zeros((npad, L), jnp.float32)

    degp = _make_deg_kernel(npad, ch_d)(col3d, ew3d, zeros_d)
    degp_t = degp[:, :n, 0].T

    blk = 2000
    g, dis = pl.pallas_call(
        _transform_body,
        grid=(n // blk,),
        in_specs=[
            pl.BlockSpec((blk, NC), lambda i: (i, 0)),
            pl.BlockSpec((blk, d_in), lambda i: (i, 0)),
            pl.BlockSpec((d_in, d), lambda i: (0, 0)),
        ],
        out_specs=[
            pl.BlockSpec((blk, d), lambda i: (i, 0)),
            pl.BlockSpec((blk, 1), lambda i: (i, 0)),
        ],
        out_shape=[
            jax.ShapeDtypeStruct((n, d), jnp.float32),
            jax.ShapeDtypeStruct((n, 1), jnp.float32),
        ],
    )(degp_t, x, W)

    p = _make_msg_kernel(n, npad, d, ch, sh)(rc3, ew3, g)

    out = pl.pallas_call(
        _combine_body,
        grid=(n // blk,),
        in_specs=[
            pl.BlockSpec((NC, blk, d), lambda i: (0, i, 0)),
            pl.BlockSpec((blk, d), lambda i: (i, 0)),
            pl.BlockSpec((blk, 1), lambda i: (i, 0)),
            pl.BlockSpec((1, d), lambda i: (0, 0)),
        ],
        out_specs=pl.BlockSpec((blk, d), lambda i: (i, 0)),
        out_shape=jax.ShapeDtypeStruct((n, d), jnp.float32),
    )(p, g, dis, b.reshape(1, d))
    return out


# speedup vs baseline: 21.5339x; 1.4324x over previous
import functools

import jax
import jax.numpy as jnp
from jax import lax
from jax.experimental import pallas as pl
from jax.experimental.pallas import tpu as pltpu
from jax.experimental.pallas import tpu_sc as plsc

NC = 2
NS = 16
NW = NC * NS
L = 16
K = 32
NG = 4
NSC = 2


def _make_deg_kernel(npad, ch_d):
    stripe = npad // NS
    kd = 128
    mesh = plsc.VectorSubcoreMesh(core_axis_name="c", subcore_axis_name="s")

    @functools.partial(
        pl.kernel,
        mesh=mesh,
        out_type=jax.ShapeDtypeStruct((NC, npad, L), jnp.float32),
        compiler_params=pltpu.CompilerParams(use_tc_tiling_on_sc=False),
        scratch_types=[
            pltpu.VMEM((ch_d, kd), jnp.int32),
            pltpu.VMEM((ch_d, kd), jnp.float32),
            pltpu.VMEM((kd, L), jnp.float32),
            pltpu.VMEM_SHARED((npad, L), jnp.float32),
        ],
    )
    def deg_k(col_hbm, ew_hbm, z_hbm, degp_hbm, col_l, ew_l, rb, acc):
        cid = lax.axis_index("c")
        sid = lax.axis_index("s")
        wid = cid * NS + sid
        base = sid * stripe
        pltpu.sync_copy(z_hbm.at[pl.ds(base, stripe)],
                        acc.at[pl.ds(base, stripe)])
        pltpu.sync_copy(col_hbm.at[wid], col_l)
        pltpu.sync_copy(ew_hbm.at[wid], ew_l)
        plsc.subcore_barrier()
        ones = jnp.ones((L,), jnp.float32)

        @pl.loop(0, ch_d)
        def _edges(g):
            for j in range(kd // L):
                ew16 = ew_l[g, pl.ds(j * L, L)]
                for lane in range(L):
                    rb[j * L + lane, pl.ds(0, L)] = ones * ew16[lane]
            pltpu.sync_copy(rb, acc.at[col_l.at[g]], add=True)

        plsc.subcore_barrier()
        pltpu.sync_copy(acc.at[pl.ds(base, stripe)],
                        degp_hbm.at[cid, pl.ds(base, stripe)])

    return deg_k


def _make_msg_kernel(npad, d, chm, ch0, ch1, sh):
    stripe = npad // NS
    msk = (1 << sh) - 1
    mesh = plsc.VectorSubcoreMesh(core_axis_name="c", subcore_axis_name="s")

    @functools.partial(
        pl.kernel,
        mesh=mesh,
        out_type=jax.ShapeDtypeStruct((NC, npad, d), jnp.float32),
        compiler_params=pltpu.CompilerParams(use_tc_tiling_on_sc=False),
        scratch_types=[
            pltpu.VMEM((chm, K), jnp.int32),
            pltpu.VMEM((NG, K), jnp.float32),
            pltpu.VMEM((NG, K), jnp.int32),
            pltpu.VMEM((NSC, K), jnp.int32),
            pltpu.VMEM_SHARED((npad, d), jnp.float32),
        ]
        + [pltpu.VMEM((K, d), jnp.float32)] * (NG + NSC)
        + [pltpu.SemaphoreType.DMA] * (2 * NG + NSC),
    )
    def msg_k(rc_hbm, ew_hbm, g_hbm, p_hbm,
              rc_l, ewb, rowb, colb, acc, *rest):
        gb = rest[:NG]
        sb = rest[NG:NG + NSC]
        sg = rest[NG + NSC:2 * NG + NSC]
        se = rest[2 * NG + NSC:3 * NG + NSC]
        ss = rest[3 * NG + NSC:]
        cid = lax.axis_index("c")
        sid = lax.axis_index("s")
        wid = cid * NS + sid
        chv = jnp.where(cid == 0, ch0, ch1)

        pltpu.sync_copy(rc_hbm.at[wid], rc_l)

        def unpack_rows(j, bg):
            for i in range(K // L):
                rowb[bg, pl.ds(i * L, L)] = lax.shift_right_logical(
                    rc_l[j, pl.ds(i * L, L)], sh)

        def unpack_cols(j, bs):
            for i in range(K // L):
                colb[bs, pl.ds(i * L, L)] = lax.bitwise_and(
                    rc_l[j, pl.ds(i * L, L)], msk)

        for b in range(NG):
            unpack_rows(b, b)
            pltpu.async_copy(g_hbm.at[rowb.at[b]], gb[b], sg[b])
            pltpu.async_copy(ew_hbm.at[wid, b], ewb.at[b], se[b])

        z = jnp.zeros((L,), jnp.float32)

        @pl.loop(0, K, unroll=4)
        def _zero(e):
            for j in range(d // L):
                sb[0][e, pl.ds(j * L, L)] = z

        base = sid * stripe
        off = 0
        while off < stripe:
            sz = min(K, stripe - off)
            pltpu.sync_copy(sb[0].at[pl.ds(0, sz)],
                            acc.at[pl.ds(base + off, sz)])
            off += sz
        plsc.subcore_barrier()

        def scale(src, dst, bg):
            @pl.loop(0, K // L)
            def _grp(i):
                ew16 = ewb[bg, pl.ds(i * L, L)]
                for lane in range(L):
                    cval = ew16[lane]
                    ei = i * L + lane
                    for j in range(d // L):
                        dst[ei, pl.ds(j * L, L)] = (
                            src[ei, pl.ds(j * L, L)] * cval)

        def chunk_body(j, bg, bs, drain_sb):
            pltpu.make_async_copy(g_hbm.at[rowb.at[bg]], gb[bg],
                                  sg[bg]).wait()
            pltpu.make_async_copy(ew_hbm.at[wid, 0], ewb.at[bg],
                                  se[bg]).wait()
            if drain_sb:
                pltpu.make_async_copy(sb[bs], acc.at[colb.at[bs]],
                                      ss[bs]).wait()
            unpack_cols(j, bs)
            scale(gb[bg], sb[bs], bg)

            @pl.when(j + NG < chv)
            def _():
                unpack_rows(j + NG, bg)
                pltpu.async_copy(g_hbm.at[rowb.at[bg]], gb[bg], sg[bg])
                pltpu.async_copy(ew_hbm.at[wid, j + NG], ewb.at[bg], se[bg])

            pltpu.async_copy(sb[bs], acc.at[colb.at[bs]], ss[bs], add=True)

        for j in range(NG):
            chunk_body(j, j % NG, j % NSC, j >= NSC)

        @pl.loop(NG, chv, step=NG)
        def _steady(g):
            for b in range(NG):
                chunk_body(g + b, b, (b % NSC), True)

        for bs in range(NSC):
            pltpu.make_async_copy(sb[bs], acc.at[colb.at[bs]], ss[bs]).wait()

        plsc.subcore_barrier()
        pltpu.sync_copy(acc.at[pl.ds(base, stripe)],
                        p_hbm.at[cid, pl.ds(base, stripe)])

    return msg_k


def _transform_body(degp_ref, x_ref, w_ref, g_ref, dis_ref):
    deg = jnp.sum(degp_ref[...], axis=1, keepdims=True) + 1.0
    dis = lax.rsqrt(deg)
    h = jnp.dot(x_ref[...], w_ref[...], preferred_element_type=jnp.float32)
    g_ref[...] = h * dis
    dis_ref[...] = dis


def _combine_body(p_ref, g_ref, dis_ref, b_ref, o_ref):
    o_ref[...] = dis_ref[...] * (p_ref[0] + p_ref[1] + g_ref[...]) + b_ref[...]


def kernel(x, edge_index, edge_attr, W, b):
    n, d_in = x.shape
    d = W.shape[1]
    e = edge_attr.shape[0]
    npad = -(-n // (8 * NS)) * (8 * NS)

    epw = NW * 128
    chb = -(-e // epw) * 128
    ch_d = chb // 128
    pad = chb * NW - e
    idt = edge_index.dtype
    sh = (n - 1).bit_length()
    assert (n << sh) < 2 ** 31
    col3d = jnp.concatenate(
        [edge_index[1], jnp.zeros((pad,), idt)]).reshape(NW, ch_d, 128)
    ew3d = jnp.concatenate(
        [edge_attr, jnp.zeros((pad,), edge_attr.dtype)]).reshape(
            NW, ch_d, 128)
    zeros_d = jnp.zeros((npad, L), jnp.float32)

    frac0 = 0.27
    tot_ch = -(-e // (NS * K))
    ch0 = max(2 * NG, (int(tot_ch * frac0) // NG) * NG)
    ch1 = -(-(tot_ch - ch0) // NG) * NG
    chm = max(ch0, ch1)
    cap0 = NS * ch0 * K
    cap1 = NS * ch1 * K
    rcflat = jnp.concatenate(
        [(edge_index[0] << sh) | edge_index[1],
         jnp.zeros((cap0 + cap1 - e,), idt)])
    ewflat = jnp.concatenate(
        [edge_attr, jnp.zeros((cap0 + cap1 - e,), edge_attr.dtype)])

    def skew(flat):
        p0 = jnp.pad(flat[:cap0].reshape(NS, ch0, K),
                     ((0, 0), (0, chm - ch0), (0, 0)))
        p1 = jnp.pad(flat[cap0:].reshape(NS, ch1, K),
                     ((0, 0), (0, chm - ch1), (0, 0)))
        return jnp.concatenate([p0, p1], axis=0)

    rc3 = skew(rcflat)
    ew3 = skew(ewflat)

    degp = _make_deg_kernel(npad, ch_d)(col3d, ew3d, zeros_d)
    degp_t = degp[:, :n, 0].T

    blk = 2000
    g, dis = pl.pallas_call(
        _transform_body,
        grid=(n // blk,),
        in_specs=[
            pl.BlockSpec((blk, NC), lambda i: (i, 0)),
            pl.BlockSpec((blk, d_in), lambda i: (i, 0)),
            pl.BlockSpec((d_in, d), lambda i: (0, 0)),
        ],
        out_specs=[
            pl.BlockSpec((blk, d), lambda i: (i, 0)),
            pl.BlockSpec((blk, 1), lambda i: (i, 0)),
        ],
        out_shape=[
            jax.ShapeDtypeStruct((n, d), jnp.float32),
            jax.ShapeDtypeStruct((n, 1), jnp.float32),
        ],
    )(degp_t, x, W)

    p = _make_msg_kernel(npad, d, chm, ch0, ch1, sh)(rc3, ew3, g)

    out = pl.pallas_call(
        _combine_body,
        grid=(n // blk,),
        in_specs=[
            pl.BlockSpec((NC, blk, d), lambda i: (0, i, 0)),
            pl.BlockSpec((blk, d), lambda i: (i, 0)),
            pl.BlockSpec((blk, 1), lambda i: (i, 0)),
            pl.BlockSpec((1, d), lambda i: (0, 0)),
        ],
        out_specs=pl.BlockSpec((blk, d), lambda i: (i, 0)),
        out_shape=jax.ShapeDtypeStruct((n, d), jnp.float32),
    )(p, g, dis, b.reshape(1, d))
    return out
